# baseline (speedup 1.0000x reference)
import jax
import jax.numpy as jnp
from jax.experimental import pallas as pl
from jax.experimental.pallas import tpu as pltpu

_NUM_BANDS = 4
_DIM = 256
_K = 1024
_CB_DIM = 128
_B = 8
_T = 1024
_BPS = 4


def _vq_body(x_ref, frozen_ref, w_ref, q_ref, idx_ref, loss_ref,
             cbm2_ref, cbhi_ref, c2_ref, ext_ref):
    band = pl.program_id(0)
    j = pl.program_id(1)

    @pl.when(j == 0)
    def _():
        cb = jax.lax.dot_general(
            frozen_ref[0], w_ref[0],
            (((1,), (1,)), ((), ())),
            preferred_element_type=jnp.float32,
        )
        cbm2_ref[...] = (-2.0 * cb).astype(jnp.bfloat16)
        cbhi_ref[...] = cb.astype(jnp.bfloat16)
        c2_ref[...] = jnp.sum(cb * cb, axis=1, keepdims=True)
        kcol = jax.lax.broadcasted_iota(jnp.int32, (_K, 8), 0)
        lane = jax.lax.broadcasted_iota(jnp.int32, (_K, 8), 1)
        vals = jnp.where(lane == 0, kcol // 32,
                         jnp.where(lane == 1, kcol % 32,
                                   jnp.where(lane == 2, 1, 0)))
        ext_ref[...] = vals.astype(jnp.bfloat16)

    @pl.when((band == 0) & (j == 0))
    def _():
        loss_ref[...] = jnp.zeros_like(loss_ref)

    scale = 1.25 / (_NUM_BANDS * _B * _T * _DIM)

    acc = jnp.zeros((1, 1), jnp.float32)
    for r in range(_BPS):
        xb = x_ref[r, 0]
        s2 = jax.lax.dot_general(
            cbm2_ref[...], xb, (((1,), (0,)), ((), ())),
            preferred_element_type=jnp.float32,
        )
        dist = s2 + c2_ref[...]
        minval = jnp.min(dist, axis=0, keepdims=True)
        eqbf = (dist == minval).astype(jnp.bfloat16)
        qT = jax.lax.dot_general(
            cbhi_ref[...], eqbf, (((0,), (0,)), ((), ())),
            preferred_element_type=jnp.float32,
        )
        ex = jax.lax.dot_general(
            ext_ref[...], eqbf, (((0,), (0,)), ((), ())),
            preferred_element_type=jnp.float32,
        )
        idxf = ex[0:1] * 32.0 + ex[1:2]
        q_ref[r, 0] = qT
        idx_ref[r, 0, 0] = idxf[0].astype(jnp.int32)

        @pl.when(jnp.max(ex[2:3]) > 1.5)
        def _(r=r, dist=dist, minval=minval):
            kiota = jax.lax.broadcasted_iota(
                jnp.int32, (_K, _T), 0).astype(jnp.float32)
            idxe = jnp.min(jnp.where(dist == minval, kiota, float(_K)),
                           axis=0, keepdims=True)
            onehot = (kiota == idxe).astype(jnp.bfloat16)
            qTe = jax.lax.dot_general(
                cbhi_ref[...], onehot, (((0,), (0,)), ((), ())),
                preferred_element_type=jnp.float32,
            )
            q_ref[r, 0] = qTe
            idx_ref[r, 0, 0] = idxe[0].astype(jnp.int32)

        x2 = jnp.sum(xb * xb, axis=0, keepdims=True)
        acc = acc + scale * jnp.sum(minval + x2)
    loss_ref[...] = loss_ref[...] + acc


def kernel(x, frozen_codebooks, Ws):
    q, idx_staged, loss = pl.pallas_call(
        _vq_body,
        grid=(_NUM_BANDS, _B // _BPS),
        in_specs=[
            pl.BlockSpec((_BPS, 1, _DIM, _T), lambda i, j: (j, i, 0, 0)),
            pl.BlockSpec((1, _K, _CB_DIM), lambda i, j: (i, 0, 0)),
            pl.BlockSpec((1, _DIM, _CB_DIM), lambda i, j: (i, 0, 0)),
        ],
        out_specs=(
            pl.BlockSpec((_BPS, 1, _DIM, _T), lambda i, j: (j, i, 0, 0)),
            pl.BlockSpec((_BPS, 1, 1, _T), lambda i, j: (j, i, 0, 0)),
            pl.BlockSpec((1, 1), lambda i, j: (0, 0)),
        ),
        out_shape=(
            jax.ShapeDtypeStruct((_B, _NUM_BANDS, _DIM, _T), jnp.float32),
            jax.ShapeDtypeStruct((_B, _NUM_BANDS, 1, _T), jnp.int32),
            jax.ShapeDtypeStruct((1, 1), jnp.float32),
        ),
        scratch_shapes=[
            pltpu.VMEM((_K, _DIM), jnp.bfloat16),
            pltpu.VMEM((_K, _DIM), jnp.bfloat16),
            pltpu.VMEM((_K, 1), jnp.float32),
            pltpu.VMEM((_K, 8), jnp.bfloat16),
        ],
        compiler_params=pltpu.CompilerParams(
            dimension_semantics=("arbitrary", "arbitrary"),
        ),
    )(x, frozen_codebooks, Ws)
    return q, idx_staged.reshape(_B, _NUM_BANDS, _T), loss[0, 0]


# speedup vs baseline: 1.1023x; 1.1023x over previous
import jax
import jax.numpy as jnp
from jax.experimental import pallas as pl
from jax.experimental.pallas import tpu as pltpu

_NUM_BANDS = 4
_DIM = 256
_K = 1024
_CB_DIM = 128
_B = 8
_T = 1024
_BPS = 4


def _vq_body(x_ref, frozen_ref, w_ref, q_ref, idx_ref, loss_ref,
             cbm2_ref, cbhi_ref, c2_ref):
    band = pl.program_id(0)
    j = pl.program_id(1)

    @pl.when(j == 0)
    def _():
        cb = jax.lax.dot_general(
            frozen_ref[0], w_ref[0],
            (((1,), (1,)), ((), ())),
            preferred_element_type=jnp.float32,
        )
        cbm2_ref[...] = (-2.0 * cb).astype(jnp.bfloat16)
        cbhi_ref[...] = cb.astype(jnp.bfloat16)
        c2_ref[...] = jnp.sum(cb * cb, axis=1, keepdims=True)

    @pl.when((band == 0) & (j == 0))
    def _():
        loss_ref[...] = jnp.zeros_like(loss_ref)

    kiota = jax.lax.broadcasted_iota(
        jnp.int32, (_K, _T), 0).astype(jnp.float32)
    scale = 1.25 / (_NUM_BANDS * _B * _T * _DIM)

    acc = jnp.zeros((1, 1), jnp.float32)
    for r in range(_BPS):
        xb = x_ref[r, 0]
        s2 = jax.lax.dot_general(
            cbm2_ref[...], xb, (((1,), (0,)), ((), ())),
            preferred_element_type=jnp.float32,
        )
        dist = s2 + c2_ref[...]
        minval = jnp.min(dist, axis=0, keepdims=True)
        idxf = jnp.min(jnp.where(dist == minval, kiota, float(_K)),
                       axis=0, keepdims=True)
        idx_ref[r, 0, 0] = idxf[0].astype(jnp.int32)
        onehot = (kiota == idxf).astype(jnp.bfloat16)
        qT = jax.lax.dot_general(
            cbhi_ref[...], onehot, (((0,), (0,)), ((), ())),
            preferred_element_type=jnp.float32,
        )
        q_ref[r, 0] = qT
        x2 = jnp.sum(xb * xb, axis=0, keepdims=True)
        acc = acc + scale * jnp.sum(minval + x2)
    loss_ref[...] = loss_ref[...] + acc


def kernel(x, frozen_codebooks, Ws):
    q, idx_staged, loss = pl.pallas_call(
        _vq_body,
        grid=(_NUM_BANDS, _B // _BPS),
        in_specs=[
            pl.BlockSpec((_BPS, 1, _DIM, _T), lambda i, j: (j, i, 0, 0)),
            pl.BlockSpec((1, _K, _CB_DIM), lambda i, j: (i, 0, 0)),
            pl.BlockSpec((1, _DIM, _CB_DIM), lambda i, j: (i, 0, 0)),
        ],
        out_specs=(
            pl.BlockSpec((_BPS, 1, _DIM, _T), lambda i, j: (j, i, 0, 0)),
            pl.BlockSpec((_BPS, 1, 1, _T), lambda i, j: (j, i, 0, 0)),
            pl.BlockSpec((1, 1), lambda i, j: (0, 0)),
        ),
        out_shape=(
            jax.ShapeDtypeStruct((_B, _NUM_BANDS, _DIM, _T), jnp.float32),
            jax.ShapeDtypeStruct((_B, _NUM_BANDS, 1, _T), jnp.int32),
            jax.ShapeDtypeStruct((1, 1), jnp.float32),
        ),
        scratch_shapes=[
            pltpu.VMEM((_K, _DIM), jnp.bfloat16),
            pltpu.VMEM((_K, _DIM), jnp.bfloat16),
            pltpu.VMEM((_K, 1), jnp.float32),
        ],
        compiler_params=pltpu.CompilerParams(
            dimension_semantics=("arbitrary", "arbitrary"),
        ),
    )(x, frozen_codebooks, Ws)
    return q, idx_staged.reshape(_B, _NUM_BANDS, _T), loss[0, 0]
